# Initial kernel scaffold; baseline (speedup 1.0000x reference)
import jax
import jax.numpy as jnp
from jax.experimental import pallas as pl


def kernel(x, edge_time, edge_index, x_time, edge_same, params):
    raise NotImplementedError("write your pallas kernel here")


# speedup vs baseline: 2.4356x; 2.4356x over previous
import functools
import math

import jax
import jax.numpy as jnp
import numpy as np
from jax.experimental import pallas as pl
from jax.experimental.pallas import tpu as pltpu

N = 10000
E = 320000
D = 128
BN = 400
BE = 2560
RSQRT_D = 1.0 / math.sqrt(D)

DIV = np.array([1.0 / np.power(10000.0, 2 * (j // 2) / D) for j in range(D)],
               dtype=np.float32).reshape(1, D)
PARITY = np.asarray((np.arange(D) % 2 == 0), dtype=np.bool_).reshape(1, D)


def _gelu(u):
    return 0.5 * u * (1.0 + jax.lax.erf(u * np.float32(1.0 / math.sqrt(2.0))))


def _dense_pre_body(h, ln_g, ln_b, wt_top, bt, wq, bq, wkdT, wksT,
                    pre, q, qtd, qts):
    hb = h[...]
    mu = jnp.mean(hb, axis=1, keepdims=True)
    var = jnp.mean((hb - mu) ** 2, axis=1, keepdims=True)
    xn = (hb - mu) * jax.lax.rsqrt(var + 1e-5) * ln_g[...] + ln_b[...]
    pre[...] = jnp.dot(xn, wt_top[...], preferred_element_type=jnp.float32) + bt[...]
    qv = jnp.dot(xn, wq[...], preferred_element_type=jnp.float32) + bq[...]
    q[...] = qv
    qtd[...] = jnp.dot(qv, wkdT[...], preferred_element_type=jnp.float32)
    qts[...] = jnp.dot(qv, wksT[...], preferred_element_type=jnp.float32)


def _dense_pre(h, p):
    wt = p['w_transfer']['w']
    row = lambda a: a.reshape(1, D)
    full = pl.BlockSpec((D, D), lambda i: (0, 0))
    rowspec = pl.BlockSpec((1, D), lambda i: (0, 0))
    blk = pl.BlockSpec((BN, D), lambda i: (i, 0))
    outs = jax.ShapeDtypeStruct((N, D), jnp.float32)
    return pl.pallas_call(
        _dense_pre_body,
        grid=(N // BN,),
        in_specs=[blk, rowspec, rowspec, full, rowspec, full, rowspec, full, full],
        out_specs=[blk, blk, blk, blk],
        out_shape=[outs, outs, outs, outs],
    )(h, row(p['ln_g']), row(p['ln_b']), wt[:D], row(p['w_transfer']['b']),
      p['w_q']['w'], row(p['w_q']['b']),
      p['w_k_diff']['w'].T, p['w_k_same']['w'].T)


def _edge_body(g1, g2, et, wlast, div, par, p_out, w_out):
    etc = et[...]
    u = g1[...] + etc * wlast[...]
    pt = (etc * 200.0) * div[...]
    te = jnp.where(par[...], jnp.sin(pt), jnp.cos(pt))
    xjt = _gelu(u) + te
    dot = jnp.sum(xjt * g2[...], axis=1, keepdims=True) * np.float32(RSQRT_D)
    w = jnp.exp(dot)
    w_out[...] = w
    p_out[...] = w * xjt


def _edge_pass(g1, g2, et, wlast):
    blk = pl.BlockSpec((BE, D), lambda i: (i, 0))
    col = pl.BlockSpec((BE, 1), lambda i: (i, 0))
    rowspec = pl.BlockSpec((1, D), lambda i: (0, 0))
    return pl.pallas_call(
        _edge_body,
        grid=(E // BE,),
        in_specs=[blk, blk, col, rowspec, rowspec, rowspec],
        out_specs=[blk, col],
        out_shape=[jax.ShapeDtypeStruct((E, D), jnp.float32),
                   jax.ShapeDtypeStruct((E, 1), jnp.float32)],
    )(g1, g2, et, wlast.reshape(1, D), jnp.asarray(DIV), jnp.asarray(PARITY))


def _dense_post_body(h, q, ud, us, wd, ws, wvd, bvd, wvs, bvs, bkd, bks,
                     out):
    qv = q[...]
    sq = np.float32(RSQRT_D)
    cd = jnp.exp(jnp.sum(qv * bkd[...], axis=1, keepdims=True) * sq)
    cs = jnp.exp(jnp.sum(qv * bks[...], axis=1, keepdims=True) * sq)
    wdv = wd[...]
    wsv = ws[...]
    num = (cd * (jnp.dot(ud[...], wvd[...], preferred_element_type=jnp.float32)
                 + wdv * bvd[...])
           + cs * (jnp.dot(us[...], wvs[...], preferred_element_type=jnp.float32)
                   + wsv * bvs[...]))
    den = cd * wdv + cs * wsv
    aggr = num / (den + 1e-16)
    out[...] = h[...] + _gelu(aggr)


def _dense_post(h, q, ud, us, wd, ws, p):
    blk = pl.BlockSpec((BN, D), lambda i: (i, 0))
    col = pl.BlockSpec((BN, 1), lambda i: (i, 0))
    full = pl.BlockSpec((D, D), lambda i: (0, 0))
    rowspec = pl.BlockSpec((1, D), lambda i: (0, 0))
    row = lambda a: a.reshape(1, D)
    return pl.pallas_call(
        _dense_post_body,
        grid=(N // BN,),
        in_specs=[blk, blk, blk, blk, col, col, full, rowspec, full, rowspec,
                  rowspec, rowspec],
        out_specs=blk,
        out_shape=jax.ShapeDtypeStruct((N, D), jnp.float32),
    )(h, q, ud, us, wd, ws,
      p['w_v_diff']['w'], row(p['w_v_diff']['b']),
      p['w_v_same']['w'], row(p['w_v_same']['b']),
      row(p['w_k_diff']['b']), row(p['w_k_same']['b']))


def _linear_body(act, x, w, b, out):
    y = jnp.dot(x[...], w[...], preferred_element_type=jnp.float32) + b[...]
    if act:
        y = jnp.maximum(y, 0.0)
    out[...] = y


def _linear(x, w, b, act):
    blk = pl.BlockSpec((BN, D), lambda i: (i, 0))
    full = pl.BlockSpec((D, D), lambda i: (0, 0))
    rowspec = pl.BlockSpec((1, D), lambda i: (0, 0))
    return pl.pallas_call(
        functools.partial(_linear_body, act),
        grid=(N // BN,),
        in_specs=[blk, full, rowspec],
        out_specs=blk,
        out_shape=jax.ShapeDtypeStruct((N, D), jnp.float32),
    )(x, w, b.reshape(1, D))


def kernel(x, edge_time, edge_index, x_time, edge_same, params):
    src = edge_index[0]
    dst = edge_index[1]
    esi = edge_same.astype(jnp.int32)
    gidx = dst + esi * N
    et = edge_time.reshape(E, 1)

    h = _linear(x, params['adapt_ws']['w'], params['adapt_ws']['b'], act=True)

    for l in range(2):
        p = params['layers'][l]
        pre, q, qtd, qts = _dense_pre(h, p)
        qt = jnp.concatenate([qtd, qts], axis=0)
        g1 = jnp.take(pre, src, axis=0)
        g2 = jnp.take(qt, gidx, axis=0)
        P, w = _edge_pass(g1, g2, et, p['w_transfer']['w'][D])
        U = jnp.zeros((2 * N, D), jnp.float32).at[gidx].add(P)
        WS = jnp.zeros((2 * N, 1), jnp.float32).at[gidx].add(w)
        h = _dense_post(h, q, U[:N], U[N:], WS[:N], WS[N:], p)

    return _linear(h, params['out_w_ode']['w'], params['out_w_ode']['b'], act=False)


# speedup vs baseline: 3.8266x; 1.5711x over previous
import functools
import math

import jax
import jax.numpy as jnp
import numpy as np
from jax import lax
from jax.experimental import pallas as pl
from jax.experimental.pallas import tpu as pltpu
from jax.experimental.pallas import tpu_sc as plsc

N = 10000
E = 320000
D = 128
BN = 400
BE = 2560
RSQRT_D = 1.0 / math.sqrt(D)

DIV = np.array([1.0 / np.power(10000.0, 2 * (j // 2) / D) for j in range(D)],
               dtype=np.float32).reshape(1, D)
PARITY = np.asarray((np.arange(D) % 2 == 0), dtype=np.bool_).reshape(1, D)


def _gelu(u):
    return 0.5 * u * (1.0 + jax.lax.erf(u * np.float32(1.0 / math.sqrt(2.0))))


def _dense_pre_body(h, ln_g, ln_b, wt_top, bt, wq, bq, wkdT, wksT,
                    pre, q, qtd, qts):
    hb = h[...]
    mu = jnp.mean(hb, axis=1, keepdims=True)
    var = jnp.mean((hb - mu) ** 2, axis=1, keepdims=True)
    xn = (hb - mu) * jax.lax.rsqrt(var + 1e-5) * ln_g[...] + ln_b[...]
    pre[...] = jnp.dot(xn, wt_top[...], preferred_element_type=jnp.float32) + bt[...]
    qv = jnp.dot(xn, wq[...], preferred_element_type=jnp.float32) + bq[...]
    q[...] = qv
    qtd[...] = jnp.dot(qv, wkdT[...], preferred_element_type=jnp.float32)
    qts[...] = jnp.dot(qv, wksT[...], preferred_element_type=jnp.float32)


def _dense_pre(h, p):
    wt = p['w_transfer']['w']
    row = lambda a: a.reshape(1, D)
    full = pl.BlockSpec((D, D), lambda i: (0, 0))
    rowspec = pl.BlockSpec((1, D), lambda i: (0, 0))
    blk = pl.BlockSpec((BN, D), lambda i: (i, 0))
    outs = jax.ShapeDtypeStruct((N, D), jnp.float32)
    return pl.pallas_call(
        _dense_pre_body,
        grid=(N // BN,),
        in_specs=[blk, rowspec, rowspec, full, rowspec, full, rowspec, full, full],
        out_specs=[blk, blk, blk, blk],
        out_shape=[outs, outs, outs, outs],
    )(h, row(p['ln_g']), row(p['ln_b']), wt[:D], row(p['w_transfer']['b']),
      p['w_q']['w'], row(p['w_q']['b']),
      p['w_k_diff']['w'].T, p['w_k_same']['w'].T)


def _edge_body(g1, g2, et, wlast, div, par, p_out, w_out):
    etc = et[...]
    u = g1[...] + etc * wlast[...]
    pt = (etc * 200.0) * div[...]
    te = jnp.where(par[...], jnp.sin(pt), jnp.cos(pt))
    xjt = _gelu(u) + te
    dot = jnp.sum(xjt * g2[...], axis=1, keepdims=True) * np.float32(RSQRT_D)
    w = jnp.exp(dot)
    w_out[...] = w
    p_out[...] = w * xjt


def _edge_pass(g1, g2, et, wlast):
    blk = pl.BlockSpec((BE, D), lambda i: (i, 0))
    col = pl.BlockSpec((BE, 1), lambda i: (i, 0))
    rowspec = pl.BlockSpec((1, D), lambda i: (0, 0))
    return pl.pallas_call(
        _edge_body,
        grid=(E // BE,),
        in_specs=[blk, blk, col, rowspec, rowspec, rowspec],
        out_specs=[blk, col],
        out_shape=[jax.ShapeDtypeStruct((E, D), jnp.float32),
                   jax.ShapeDtypeStruct((E, 1), jnp.float32)],
    )(g1, g2, et, wlast.reshape(1, D), jnp.asarray(DIV), jnp.asarray(PARITY))


def _dense_post_body(h, q, ud, us, wd, ws, wvd, bvd, wvs, bvs, bkd, bks,
                     out):
    qv = q[...]
    sq = np.float32(RSQRT_D)
    cd = jnp.exp(jnp.sum(qv * bkd[...], axis=1, keepdims=True) * sq)
    cs = jnp.exp(jnp.sum(qv * bks[...], axis=1, keepdims=True) * sq)
    wdv = wd[...]
    wsv = ws[...]
    num = (cd * (jnp.dot(ud[...], wvd[...], preferred_element_type=jnp.float32)
                 + wdv * bvd[...])
           + cs * (jnp.dot(us[...], wvs[...], preferred_element_type=jnp.float32)
                   + wsv * bvs[...]))
    den = cd * wdv + cs * wsv
    aggr = num / (den + 1e-16)
    out[...] = h[...] + _gelu(aggr)


def _dense_post(h, q, ud, us, wd, ws, p):
    blk = pl.BlockSpec((BN, D), lambda i: (i, 0))
    col = pl.BlockSpec((BN, 1), lambda i: (i, 0))
    full = pl.BlockSpec((D, D), lambda i: (0, 0))
    rowspec = pl.BlockSpec((1, D), lambda i: (0, 0))
    row = lambda a: a.reshape(1, D)
    return pl.pallas_call(
        _dense_post_body,
        grid=(N // BN,),
        in_specs=[blk, blk, blk, blk, col, col, full, rowspec, full, rowspec,
                  rowspec, rowspec],
        out_specs=blk,
        out_shape=jax.ShapeDtypeStruct((N, D), jnp.float32),
    )(h, q, ud, us, wd, ws,
      p['w_v_diff']['w'], row(p['w_v_diff']['b']),
      p['w_v_same']['w'], row(p['w_v_same']['b']),
      row(p['w_k_diff']['b']), row(p['w_k_same']['b']))


def _linear_body(act, x, w, b, out):
    y = jnp.dot(x[...], w[...], preferred_element_type=jnp.float32) + b[...]
    if act:
        y = jnp.maximum(y, 0.0)
    out[...] = y


def _linear(x, w, b, act):
    blk = pl.BlockSpec((BN, D), lambda i: (i, 0))
    full = pl.BlockSpec((D, D), lambda i: (0, 0))
    rowspec = pl.BlockSpec((1, D), lambda i: (0, 0))
    return pl.pallas_call(
        functools.partial(_linear_body, act),
        grid=(N // BN,),
        in_specs=[blk, full, rowspec],
        out_specs=blk,
        out_shape=jax.ShapeDtypeStruct((N, D), jnp.float32),
    )(x, w, b.reshape(1, D))


NW = 32
EPW = E // NW
GC = 400


def _sc_gather_body(pre_hbm, qt_hbm, src_hbm, gidx_hbm, g1_hbm, g2_hbm,
                    idx1, idx2, rows1, rows2, sem1, sem2):
    wid = lax.axis_index("s") * 2 + lax.axis_index("c")
    base = wid * EPW

    def chunk(i, _):
        off = base + i * GC
        pltpu.sync_copy(src_hbm.at[pl.ds(off, GC)], idx1)
        pltpu.sync_copy(gidx_hbm.at[pl.ds(off, GC)], idx2)
        a = pltpu.async_copy(pre_hbm.at[idx1], rows1, sem1)
        b = pltpu.async_copy(qt_hbm.at[idx2], rows2, sem2)
        a.wait()
        b.wait()
        pltpu.sync_copy(rows1, g1_hbm.at[pl.ds(off, GC)])
        pltpu.sync_copy(rows2, g2_hbm.at[pl.ds(off, GC)])
        return 0

    lax.fori_loop(0, EPW // GC, chunk, 0)


def _sc_gather(pre, qt, src, gidx):
    mesh = plsc.VectorSubcoreMesh(core_axis_name="c", subcore_axis_name="s",
                                  num_cores=2, num_subcores=16)
    f = pl.kernel(
        _sc_gather_body,
        mesh=mesh,
        out_type=[jax.ShapeDtypeStruct((E, D), jnp.float32),
                  jax.ShapeDtypeStruct((E, D), jnp.float32)],
        scratch_types=[pltpu.VMEM((GC,), jnp.int32),
                       pltpu.VMEM((GC,), jnp.int32),
                       pltpu.VMEM((GC, D), jnp.float32),
                       pltpu.VMEM((GC, D), jnp.float32),
                       pltpu.SemaphoreType.DMA,
                       pltpu.SemaphoreType.DMA],
    )
    return f(pre, qt, src, gidx)


def kernel(x, edge_time, edge_index, x_time, edge_same, params):
    src = edge_index[0]
    dst = edge_index[1]
    esi = edge_same.astype(jnp.int32)
    gidx = dst + esi * N
    et = edge_time.reshape(E, 1)

    h = _linear(x, params['adapt_ws']['w'], params['adapt_ws']['b'], act=True)

    for l in range(2):
        p = params['layers'][l]
        pre, q, qtd, qts = _dense_pre(h, p)
        qt = jnp.concatenate([qtd, qts], axis=0)
        g1, g2 = _sc_gather(pre, qt, src, gidx)
        P, w = _edge_pass(g1, g2, et, p['w_transfer']['w'][D])
        U = jnp.zeros((2 * N, D), jnp.float32).at[gidx].add(P)
        WS = jnp.zeros((2 * N, 1), jnp.float32).at[gidx].add(w)
        h = _dense_post(h, q, U[:N], U[N:], WS[:N], WS[N:], p)

    return _linear(h, params['out_w_ode']['w'], params['out_w_ode']['b'], act=False)
